# baseline (speedup 1.0000x reference)
import functools

import jax
import jax.numpy as jnp
from jax import lax
from jax.experimental import pallas as pl
from jax.experimental.pallas import tpu as pltpu
from jax.experimental.pallas import tpu_sc as plsc

N = 3136
C = 768
E = 8
DFF = 3072
T = 128
NB = 34
NPAD = NB * T
DT = 768
ND = DFF // DT
RB = 392
RG = N // RB
NW = 32
N_SRC = 3328
CH = N_SRC // NW
NMETA = 40
DUMMY = 4152


def _dispatch_kernel(x_ref, wr_ref, fm_ref, tm_ref, cm_ref,
                     probs_ref, dest_ref, meta_ref,
                     carry_ref, oh_ref, rank_ref):
    b = pl.program_id(0)

    @pl.when(b == 0)
    def _():
        carry_ref[...] = jnp.zeros_like(carry_ref)

    @pl.when(b < RG)
    def _():
        fp = jnp.dot(x_ref[...], fm_ref[...],
                     preferred_element_type=jnp.float32)
        tp = jnp.dot(wr_ref[...], tm_ref[...],
                     preferred_element_type=jnp.float32)
        comb = jnp.concatenate([fp, tp], axis=1)
        logits = jnp.dot(comb, cm_ref[...],
                         preferred_element_type=jnp.float32)
        m = jnp.max(logits, axis=1, keepdims=True)
        col = lax.broadcasted_iota(jnp.int32, (RB, E), 1)
        cand = jnp.where(logits == m, col, E)
        top = jnp.min(cand, axis=1, keepdims=True)
        oh = (col == top).astype(jnp.float32)
        probs_ref[...] = oh
        ri = lax.broadcasted_iota(jnp.int32, (RB, RB), 0)
        rj = lax.broadcasted_iota(jnp.int32, (RB, RB), 1)
        tri = (ri > rj).astype(jnp.float32)
        cum = jnp.dot(tri, oh, preferred_element_type=jnp.float32)
        carry = carry_ref[...]
        oh_ref[pl.ds(b * RB, RB), :] = oh
        rank_ref[pl.ds(b * RB, RB), :] = jnp.sum(oh * (cum + carry), axis=1,
                                                 keepdims=True)
        carry_ref[...] = carry + jnp.sum(oh, axis=0, keepdims=True)

    @pl.when(b == RG)
    def _():
        c = carry_ref[...].astype(jnp.int32)
        pc = (((c + (T - 1)) // T) * T).astype(jnp.float32)
        pcm = jnp.broadcast_to(pc, (E, E))
        i8 = lax.broadcasted_iota(jnp.int32, (E, E), 0)
        j8 = lax.broadcasted_iota(jnp.int32, (E, E), 1)
        poff_col = jnp.sum(jnp.where(j8 < i8, pcm, 0.0), axis=1,
                           keepdims=True)
        cum_row = jnp.dot(pc, (i8 <= j8).astype(jnp.float32),
                          preferred_element_type=jnp.float32)
        dest = jnp.dot(oh_ref[...], poff_col,
                       preferred_element_type=jnp.float32) + rank_ref[...]
        dest_ref[pl.ds(0, N), :] = dest.astype(jnp.int32)
        dest_ref[pl.ds(N, N_SRC - N), :] = DUMMY + lax.broadcasted_iota(
            jnp.int32, (N_SRC - N, 1), 0)
        thr = (lax.broadcasted_iota(jnp.int32, (NMETA, 1), 0)
               * T).astype(jnp.float32)
        be = jnp.sum((jnp.broadcast_to(cum_row, (NMETA, E)) <=
                      jnp.broadcast_to(thr, (NMETA, E))).astype(jnp.float32),
                     axis=1, keepdims=True)
        be = jnp.minimum(be, float(E - 1))
        nact = jnp.broadcast_to(cum_row[:, E - 1:] * (1.0 / T), (NMETA, 1))
        rows = lax.broadcasted_iota(jnp.int32, (NMETA, 1), 0)
        meta_ref[...] = jnp.where(rows < NB, be, nact).astype(jnp.int32)


def _ffn_kernel(meta_ref, x_ref, w1_ref, b1_ref, w2_ref, b2_ref, o_ref):
    bb = pl.program_id(0)
    j = pl.program_id(1)
    nact = meta_ref[NB, 0]

    @pl.when(bb < nact)
    def _():
        h = jnp.dot(x_ref[...], w1_ref[0], preferred_element_type=jnp.float32)
        h = h + b1_ref[0]
        g = h * 0.5 * (1.0 + lax.erf(h * 0.7071067811865476))
        contrib = jnp.dot(g, w2_ref[0], preferred_element_type=jnp.float32)

        @pl.when(j == 0)
        def _():
            o_ref[...] = jnp.broadcast_to(b2_ref[0], (T, C))

        o_ref[...] += contrib


@functools.cache
def _sc_kernels():
    mesh = plsc.VectorSubcoreMesh(core_axis_name="c", subcore_axis_name="s")

    @functools.partial(
        pl.kernel, mesh=mesh,
        out_type=jax.ShapeDtypeStruct((NPAD, C), jnp.float32),
        scratch_types=[pltpu.VMEM((CH,), jnp.int32),
                       pltpu.VMEM((CH, C), jnp.float32),
                       pltpu.SemaphoreType.DMA],
    )
    def sc_scatter(src_hbm, idx_hbm, out_hbm, idx_v, rows_v, sem):
        wid = lax.axis_index("s") * 2 + lax.axis_index("c")
        base = wid * CH
        pltpu.sync_copy(src_hbm.at[pl.ds(base, CH)], rows_v)
        pltpu.sync_copy(idx_hbm.at[pl.ds(base, CH)], idx_v)
        pltpu.async_copy(rows_v, out_hbm.at[idx_v], sem).wait()

    @functools.partial(
        pl.kernel, mesh=mesh,
        out_type=jax.ShapeDtypeStruct((N_SRC, C), jnp.float32),
        scratch_types=[pltpu.VMEM((CH,), jnp.int32),
                       pltpu.VMEM((CH, C), jnp.float32),
                       pltpu.SemaphoreType.DMA],
    )
    def sc_gather(src_hbm, idx_hbm, out_hbm, idx_v, rows_v, sem):
        wid = lax.axis_index("s") * 2 + lax.axis_index("c")
        base = wid * CH
        pltpu.sync_copy(idx_hbm.at[pl.ds(base, CH)], idx_v)
        pltpu.async_copy(src_hbm.at[idx_v], rows_v, sem).wait()
        pltpu.sync_copy(rows_v, out_hbm.at[pl.ds(base, CH)])

    return sc_scatter, sc_gather


def kernel(x, w, feature_mu, text_mu, combined_mu, temperature, W1, b1, W2, b2):
    del temperature
    Bc, Cc, Hh, Ww = x.shape
    x_tok = jnp.transpose(x, (0, 2, 3, 1)).reshape(N, C)
    w_r = jnp.broadcast_to(w[:, None, None, :],
                           (Bc, Hh, Ww, w.shape[1])).reshape(N, w.shape[1])

    probs, dest, meta = pl.pallas_call(
        _dispatch_kernel,
        grid=(RG + 1,),
        in_specs=[
            pl.BlockSpec((RB, C), lambda b: (jnp.minimum(b, RG - 1), 0)),
            pl.BlockSpec((RB, 512), lambda b: (jnp.minimum(b, RG - 1), 0)),
            pl.BlockSpec((C, 128), lambda b: (0, 0)),
            pl.BlockSpec((512, 128), lambda b: (0, 0)),
            pl.BlockSpec((256, E), lambda b: (0, 0)),
        ],
        out_specs=[
            pl.BlockSpec((RB, E), lambda b: (jnp.minimum(b, RG - 1), 0)),
            pl.BlockSpec((N_SRC, 1), lambda b: (0, 0)),
            pl.BlockSpec((NMETA, 1), lambda b: (0, 0)),
        ],
        out_shape=[
            jax.ShapeDtypeStruct((N, E), jnp.float32),
            jax.ShapeDtypeStruct((N_SRC, 1), jnp.int32),
            jax.ShapeDtypeStruct((NMETA, 1), jnp.int32),
        ],
        scratch_shapes=[pltpu.VMEM((1, E), jnp.float32),
                        pltpu.VMEM((N, E), jnp.float32),
                        pltpu.VMEM((N, 1), jnp.float32)],
    )(x_tok, w_r, feature_mu, text_mu, combined_mu)

    dest_flat = dest.reshape(N_SRC)
    x_src = jnp.pad(x_tok, ((0, N_SRC - N), (0, 0)))
    x_grouped = jnp.zeros((NPAD, C), jnp.float32).at[dest_flat].set(x_src)

    y_grouped = pl.pallas_call(
        _ffn_kernel,
        grid_spec=pltpu.PrefetchScalarGridSpec(
            num_scalar_prefetch=1,
            grid=(NB, ND),
            in_specs=[
                pl.BlockSpec((T, C), lambda bb, j, s: (bb, 0)),
                pl.BlockSpec((1, C, DT), lambda bb, j, s: (s[bb, 0], 0, j)),
                pl.BlockSpec((1, 1, DT),
                             lambda bb, j, s: (s[bb, 0] * ND + j, 0, 0)),
                pl.BlockSpec((1, DT, C), lambda bb, j, s: (s[bb, 0], j, 0)),
                pl.BlockSpec((1, 1, C), lambda bb, j, s: (s[bb, 0], 0, 0)),
            ],
            out_specs=pl.BlockSpec((T, C), lambda bb, j, s: (bb, 0)),
        ),
        out_shape=jax.ShapeDtypeStruct((NPAD, C), jnp.float32),
    )(meta, x_grouped, W1, b1.reshape(E * ND, 1, DT), W2,
      b2.reshape(E, 1, C))

    out_tok = y_grouped[dest_flat[:N]]
    output = jnp.transpose(out_tok.reshape(Bc, Hh, Ww, Cc), (0, 3, 1, 2))
    kl_div = jnp.asarray(0.0, dtype=jnp.float32)
    return (output, kl_div, probs)


# speedup vs baseline: 1.6033x; 1.6033x over previous
import functools

import jax
import jax.numpy as jnp
from jax import lax
from jax.experimental import pallas as pl
from jax.experimental.pallas import tpu as pltpu
from jax.experimental.pallas import tpu_sc as plsc

N = 3136
C = 768
E = 8
DFF = 3072
T = 128
NB = 34
NPAD = NB * T
DT = 768
ND = DFF // DT
RB = 392
RG = N // RB
NW = 32
N_SRC = 3328
CH = N_SRC // NW
NMETA = 40
DUMMY = 4152


def _dispatch_kernel(x_ref, wr_ref, fm_ref, tm_ref, cm_ref,
                     probs_ref, dest_ref, meta_ref,
                     carry_ref, oh_ref, rank_ref):
    b = pl.program_id(0)

    @pl.when(b == 0)
    def _():
        carry_ref[...] = jnp.zeros_like(carry_ref)

    @pl.when(b < RG)
    def _():
        fp = jnp.dot(x_ref[...], fm_ref[...],
                     preferred_element_type=jnp.float32)
        tp = jnp.dot(wr_ref[...], tm_ref[...],
                     preferred_element_type=jnp.float32)
        comb = jnp.concatenate([fp, tp], axis=1)
        logits = jnp.dot(comb, cm_ref[...],
                         preferred_element_type=jnp.float32)
        m = jnp.max(logits, axis=1, keepdims=True)
        col = lax.broadcasted_iota(jnp.int32, (RB, E), 1)
        cand = jnp.where(logits == m, col, E)
        top = jnp.min(cand, axis=1, keepdims=True)
        oh = (col == top).astype(jnp.float32)
        probs_ref[...] = oh
        ri = lax.broadcasted_iota(jnp.int32, (RB, RB), 0)
        rj = lax.broadcasted_iota(jnp.int32, (RB, RB), 1)
        tri = (ri > rj).astype(jnp.float32)
        cum = jnp.dot(tri, oh, preferred_element_type=jnp.float32)
        carry = carry_ref[...]
        oh_ref[pl.ds(b * RB, RB), :] = oh
        rank_ref[pl.ds(b * RB, RB), :] = jnp.sum(oh * (cum + carry), axis=1,
                                                 keepdims=True)
        carry_ref[...] = carry + jnp.sum(oh, axis=0, keepdims=True)

    @pl.when(b == RG)
    def _():
        c = carry_ref[...].astype(jnp.int32)
        pc = (((c + (T - 1)) // T) * T).astype(jnp.float32)
        pcm = jnp.broadcast_to(pc, (E, E))
        i8 = lax.broadcasted_iota(jnp.int32, (E, E), 0)
        j8 = lax.broadcasted_iota(jnp.int32, (E, E), 1)
        poff_col = jnp.sum(jnp.where(j8 < i8, pcm, 0.0), axis=1,
                           keepdims=True)
        cum_row = jnp.dot(pc, (i8 <= j8).astype(jnp.float32),
                          preferred_element_type=jnp.float32)
        dest = jnp.dot(oh_ref[...], poff_col,
                       preferred_element_type=jnp.float32) + rank_ref[...]
        dest_ref[pl.ds(0, N), :] = dest.astype(jnp.int32)
        dest_ref[pl.ds(N, N_SRC - N), :] = DUMMY + lax.broadcasted_iota(
            jnp.int32, (N_SRC - N, 1), 0)
        thr = (lax.broadcasted_iota(jnp.int32, (NMETA, 1), 0)
               * T).astype(jnp.float32)
        be = jnp.sum((jnp.broadcast_to(cum_row, (NMETA, E)) <=
                      jnp.broadcast_to(thr, (NMETA, E))).astype(jnp.float32),
                     axis=1, keepdims=True)
        be = jnp.minimum(be, float(E - 1))
        nact = jnp.broadcast_to(cum_row[:, E - 1:] * (1.0 / T), (NMETA, 1))
        rows = lax.broadcasted_iota(jnp.int32, (NMETA, 1), 0)
        meta_ref[...] = jnp.where(rows < NB, be, nact).astype(jnp.int32)


def _ffn_kernel(meta_ref, x_ref, w1_ref, b1_ref, w2_ref, b2_ref, o_ref):
    bb = pl.program_id(0)
    nact = meta_ref[NB, 0]

    @pl.when(bb < nact)
    def _():
        h = jnp.dot(x_ref[...], w1_ref[0], preferred_element_type=jnp.float32)
        h = h + b1_ref[0]
        g = h * 0.5 * (1.0 + lax.erf(h * 0.7071067811865476))
        o_ref[...] = (jnp.dot(g, w2_ref[0], preferred_element_type=jnp.float32)
                      + b2_ref[0])


@functools.cache
def _sc_kernels():
    mesh = plsc.VectorSubcoreMesh(core_axis_name="c", subcore_axis_name="s")

    @functools.partial(
        pl.kernel, mesh=mesh,
        out_type=jax.ShapeDtypeStruct((NPAD, C), jnp.float32),
        scratch_types=[pltpu.VMEM((CH,), jnp.int32),
                       pltpu.VMEM((CH, C), jnp.float32),
                       pltpu.SemaphoreType.DMA],
    )
    def sc_scatter(src_hbm, idx_hbm, out_hbm, idx_v, rows_v, sem):
        wid = lax.axis_index("s") * 2 + lax.axis_index("c")
        base = wid * CH
        pltpu.sync_copy(src_hbm.at[pl.ds(base, CH)], rows_v)
        pltpu.sync_copy(idx_hbm.at[pl.ds(base, CH)], idx_v)
        pltpu.async_copy(rows_v, out_hbm.at[idx_v], sem).wait()

    @functools.partial(
        pl.kernel, mesh=mesh,
        out_type=jax.ShapeDtypeStruct((N_SRC, C), jnp.float32),
        scratch_types=[pltpu.VMEM((CH,), jnp.int32),
                       pltpu.VMEM((CH, C), jnp.float32),
                       pltpu.SemaphoreType.DMA],
    )
    def sc_gather(src_hbm, idx_hbm, out_hbm, idx_v, rows_v, sem):
        wid = lax.axis_index("s") * 2 + lax.axis_index("c")
        base = wid * CH
        pltpu.sync_copy(idx_hbm.at[pl.ds(base, CH)], idx_v)
        pltpu.async_copy(src_hbm.at[idx_v], rows_v, sem).wait()
        pltpu.sync_copy(rows_v, out_hbm.at[pl.ds(base, CH)])

    return sc_scatter, sc_gather


def kernel(x, w, feature_mu, text_mu, combined_mu, temperature, W1, b1, W2, b2):
    del temperature
    Bc, Cc, Hh, Ww = x.shape
    x_tok = jnp.transpose(x, (0, 2, 3, 1)).reshape(N, C)
    w_r = jnp.broadcast_to(w[:, None, None, :],
                           (Bc, Hh, Ww, w.shape[1])).reshape(N, w.shape[1])

    probs, dest, meta = pl.pallas_call(
        _dispatch_kernel,
        grid=(RG + 1,),
        in_specs=[
            pl.BlockSpec((RB, C), lambda b: (jnp.minimum(b, RG - 1), 0)),
            pl.BlockSpec((RB, 512), lambda b: (jnp.minimum(b, RG - 1), 0)),
            pl.BlockSpec((C, 128), lambda b: (0, 0)),
            pl.BlockSpec((512, 128), lambda b: (0, 0)),
            pl.BlockSpec((256, E), lambda b: (0, 0)),
        ],
        out_specs=[
            pl.BlockSpec((RB, E), lambda b: (jnp.minimum(b, RG - 1), 0)),
            pl.BlockSpec((N_SRC, 1), lambda b: (0, 0)),
            pl.BlockSpec((NMETA, 1), lambda b: (0, 0)),
        ],
        out_shape=[
            jax.ShapeDtypeStruct((N, E), jnp.float32),
            jax.ShapeDtypeStruct((N_SRC, 1), jnp.int32),
            jax.ShapeDtypeStruct((NMETA, 1), jnp.int32),
        ],
        scratch_shapes=[pltpu.VMEM((1, E), jnp.float32),
                        pltpu.VMEM((N, E), jnp.float32),
                        pltpu.VMEM((N, 1), jnp.float32)],
    )(x_tok, w_r, feature_mu, text_mu, combined_mu)

    sc_scatter, sc_gather = _sc_kernels()
    dest_flat = dest.reshape(N_SRC)
    x_src = jnp.pad(x_tok, ((0, N_SRC - N), (0, 0)))
    x_grouped = sc_scatter(x_src, dest_flat)

    y_grouped = pl.pallas_call(
        _ffn_kernel,
        grid_spec=pltpu.PrefetchScalarGridSpec(
            num_scalar_prefetch=1,
            grid=(NB,),
            in_specs=[
                pl.BlockSpec((T, C), lambda bb, s: (bb, 0)),
                pl.BlockSpec((1, C, DFF), lambda bb, s: (s[bb, 0], 0, 0)),
                pl.BlockSpec((1, 1, DFF), lambda bb, s: (s[bb, 0], 0, 0)),
                pl.BlockSpec((1, DFF, C), lambda bb, s: (s[bb, 0], 0, 0)),
                pl.BlockSpec((1, 1, C), lambda bb, s: (s[bb, 0], 0, 0)),
            ],
            out_specs=pl.BlockSpec((T, C), lambda bb, s: (bb, 0)),
        ),
        out_shape=jax.ShapeDtypeStruct((NPAD, C), jnp.float32),
    )(meta, x_grouped, W1, b1.reshape(E, 1, DFF), W2,
      b2.reshape(E, 1, C))

    out_pad = sc_gather(y_grouped, dest_flat)
    out_tok = out_pad[:N]
    output = jnp.transpose(out_tok.reshape(Bc, Hh, Ww, Cc), (0, 3, 1, 2))
    kl_div = jnp.asarray(0.0, dtype=jnp.float32)
    return (output, kl_div, probs)


# speedup vs baseline: 1.7284x; 1.0781x over previous
import functools

import jax
import jax.numpy as jnp
from jax import lax
from jax.experimental import pallas as pl
from jax.experimental.pallas import tpu as pltpu
from jax.experimental.pallas import tpu_sc as plsc

N = 3136
C = 768
E = 8
DFF = 3072
T = 128
NB = 34
NPAD = NB * T
DT = 768
ND = DFF // DT
RB = 392
RG = N // RB
NW = 32
CH = 104
NMETA = 40


def _dispatch_kernel(x_ref, wr_ref, fm_ref, tm_ref, cm_ref,
                     probs_ref, dest_ref, meta_ref,
                     carry_ref, oh_ref, rank_ref):
    b = pl.program_id(0)

    @pl.when(b == 0)
    def _():
        carry_ref[...] = jnp.zeros_like(carry_ref)

    @pl.when(b < RG)
    def _():
        fp = jnp.dot(x_ref[...], fm_ref[...],
                     preferred_element_type=jnp.float32)
        tp = jnp.dot(wr_ref[...], tm_ref[...],
                     preferred_element_type=jnp.float32)
        comb = jnp.concatenate([fp, tp], axis=1)
        logits = jnp.dot(comb, cm_ref[...],
                         preferred_element_type=jnp.float32)
        m = jnp.max(logits, axis=1, keepdims=True)
        col = lax.broadcasted_iota(jnp.int32, (RB, E), 1)
        cand = jnp.where(logits == m, col, E)
        top = jnp.min(cand, axis=1, keepdims=True)
        oh = (col == top).astype(jnp.float32)
        probs_ref[...] = oh
        ri = lax.broadcasted_iota(jnp.int32, (RB, RB), 0)
        rj = lax.broadcasted_iota(jnp.int32, (RB, RB), 1)
        tri = (ri > rj).astype(jnp.float32)
        cum = jnp.dot(tri, oh, preferred_element_type=jnp.float32)
        carry = carry_ref[...]
        oh_ref[pl.ds(b * RB, RB), :] = oh
        rank_ref[pl.ds(b * RB, RB), :] = jnp.sum(oh * (cum + carry), axis=1,
                                                 keepdims=True)
        carry_ref[...] = carry + jnp.sum(oh, axis=0, keepdims=True)

    @pl.when(b == RG)
    def _():
        c = carry_ref[...].astype(jnp.int32)
        pc = (((c + (T - 1)) // T) * T).astype(jnp.float32)
        pcm = jnp.broadcast_to(pc, (E, E))
        i8 = lax.broadcasted_iota(jnp.int32, (E, E), 0)
        j8 = lax.broadcasted_iota(jnp.int32, (E, E), 1)
        poff_col = jnp.sum(jnp.where(j8 < i8, pcm, 0.0), axis=1,
                           keepdims=True)
        cum_row = jnp.dot(pc, (i8 <= j8).astype(jnp.float32),
                          preferred_element_type=jnp.float32)
        dest = jnp.dot(oh_ref[...], poff_col,
                       preferred_element_type=jnp.float32) + rank_ref[...]
        dest_ref[...] = dest.astype(jnp.int32)
        thr = (lax.broadcasted_iota(jnp.int32, (NMETA, 1), 0)
               * T).astype(jnp.float32)
        be = jnp.sum((jnp.broadcast_to(cum_row, (NMETA, E)) <=
                      jnp.broadcast_to(thr, (NMETA, E))).astype(jnp.float32),
                     axis=1, keepdims=True)
        be = jnp.minimum(be, float(E - 1))
        nact = jnp.broadcast_to(cum_row[:, E - 1:] * (1.0 / T), (NMETA, 1))
        rows = lax.broadcasted_iota(jnp.int32, (NMETA, 1), 0)
        meta_ref[...] = jnp.where(rows < NB, be, nact).astype(jnp.int32)


def _ffn_kernel(meta_ref, x_ref, w1_ref, b1_ref, w2_ref, b2_ref, o_ref):
    bb = pl.program_id(0)
    nact = meta_ref[NB, 0]

    @pl.when(bb < nact)
    def _():
        h = jnp.dot(x_ref[...], w1_ref[0], preferred_element_type=jnp.float32)
        h = h + b1_ref[0]
        g = h * 0.5 * (1.0 + lax.erf(h * 0.7071067811865476))
        o_ref[...] = (jnp.dot(g, w2_ref[0], preferred_element_type=jnp.float32)
                      + b2_ref[0])


@functools.cache
def _sc_kernels():
    mesh = plsc.VectorSubcoreMesh(core_axis_name="c", subcore_axis_name="s")
    TAIL = N - 30 * CH

    @functools.partial(
        pl.kernel, mesh=mesh,
        out_type=jax.ShapeDtypeStruct((NPAD, C), jnp.float32),
        scratch_types=[pltpu.VMEM((CH,), jnp.int32),
                       pltpu.VMEM((CH, C), jnp.float32),
                       pltpu.VMEM((TAIL,), jnp.int32),
                       pltpu.VMEM((TAIL, C), jnp.float32),
                       pltpu.SemaphoreType.DMA],
    )
    def sc_scatter(src_hbm, idx_hbm, out_hbm, idx_v, rows_v, idx_t, rows_t,
                   sem):
        wid = lax.axis_index("s") * 2 + lax.axis_index("c")
        base = wid * CH

        @pl.when(wid < 30)
        def _():
            pltpu.sync_copy(src_hbm.at[pl.ds(base, CH)], rows_v)
            pltpu.sync_copy(idx_hbm.at[pl.ds(base, CH)], idx_v)
            pltpu.async_copy(rows_v, out_hbm.at[idx_v], sem).wait()

        @pl.when(wid == 30)
        def _():
            pltpu.sync_copy(src_hbm.at[pl.ds(30 * CH, TAIL)], rows_t)
            pltpu.sync_copy(idx_hbm.at[pl.ds(30 * CH, TAIL)], idx_t)
            pltpu.async_copy(rows_t, out_hbm.at[idx_t], sem).wait()

    @functools.partial(
        pl.kernel, mesh=mesh,
        out_type=jax.ShapeDtypeStruct((N, C), jnp.float32),
        scratch_types=[pltpu.VMEM((CH,), jnp.int32),
                       pltpu.VMEM((CH, C), jnp.float32),
                       pltpu.VMEM((TAIL,), jnp.int32),
                       pltpu.VMEM((TAIL, C), jnp.float32),
                       pltpu.SemaphoreType.DMA],
    )
    def sc_gather(src_hbm, idx_hbm, out_hbm, idx_v, rows_v, idx_t, rows_t,
                  sem):
        wid = lax.axis_index("s") * 2 + lax.axis_index("c")
        base = wid * CH

        @pl.when(wid < 30)
        def _():
            pltpu.sync_copy(idx_hbm.at[pl.ds(base, CH)], idx_v)
            pltpu.async_copy(src_hbm.at[idx_v], rows_v, sem).wait()
            pltpu.sync_copy(rows_v, out_hbm.at[pl.ds(base, CH)])

        @pl.when(wid == 30)
        def _():
            pltpu.sync_copy(idx_hbm.at[pl.ds(30 * CH, TAIL)], idx_t)
            pltpu.async_copy(src_hbm.at[idx_t], rows_t, sem).wait()
            pltpu.sync_copy(rows_t, out_hbm.at[pl.ds(30 * CH, TAIL)])

    return sc_scatter, sc_gather


def kernel(x, w, feature_mu, text_mu, combined_mu, temperature, W1, b1, W2, b2):
    del temperature
    Bc, Cc, Hh, Ww = x.shape
    x_tok = jnp.transpose(x, (0, 2, 3, 1)).reshape(N, C)
    w_r = jnp.broadcast_to(w[:, None, None, :],
                           (Bc, Hh, Ww, w.shape[1])).reshape(N, w.shape[1])

    probs, dest, meta = pl.pallas_call(
        _dispatch_kernel,
        grid=(RG + 1,),
        in_specs=[
            pl.BlockSpec((RB, C), lambda b: (jnp.minimum(b, RG - 1), 0)),
            pl.BlockSpec((RB, 512), lambda b: (jnp.minimum(b, RG - 1), 0)),
            pl.BlockSpec((C, 128), lambda b: (0, 0)),
            pl.BlockSpec((512, 128), lambda b: (0, 0)),
            pl.BlockSpec((256, E), lambda b: (0, 0)),
        ],
        out_specs=[
            pl.BlockSpec((RB, E), lambda b: (jnp.minimum(b, RG - 1), 0)),
            pl.BlockSpec((N, 1), lambda b: (0, 0)),
            pl.BlockSpec((NMETA, 1), lambda b: (0, 0)),
        ],
        out_shape=[
            jax.ShapeDtypeStruct((N, E), jnp.float32),
            jax.ShapeDtypeStruct((N, 1), jnp.int32),
            jax.ShapeDtypeStruct((NMETA, 1), jnp.int32),
        ],
        scratch_shapes=[pltpu.VMEM((1, E), jnp.float32),
                        pltpu.VMEM((N, E), jnp.float32),
                        pltpu.VMEM((N, 1), jnp.float32)],
    )(x_tok, w_r, feature_mu, text_mu, combined_mu)

    sc_scatter, sc_gather = _sc_kernels()
    dest_flat = dest.reshape(N)
    x_grouped = sc_scatter(x_tok, dest_flat)

    y_grouped = pl.pallas_call(
        _ffn_kernel,
        grid_spec=pltpu.PrefetchScalarGridSpec(
            num_scalar_prefetch=1,
            grid=(NB,),
            in_specs=[
                pl.BlockSpec((T, C), lambda bb, s: (bb, 0)),
                pl.BlockSpec((1, C, DFF), lambda bb, s: (s[bb, 0], 0, 0)),
                pl.BlockSpec((1, 1, DFF), lambda bb, s: (s[bb, 0], 0, 0)),
                pl.BlockSpec((1, DFF, C), lambda bb, s: (s[bb, 0], 0, 0)),
                pl.BlockSpec((1, 1, C), lambda bb, s: (s[bb, 0], 0, 0)),
            ],
            out_specs=pl.BlockSpec((T, C), lambda bb, s: (bb, 0)),
        ),
        out_shape=jax.ShapeDtypeStruct((NPAD, C), jnp.float32),
    )(meta, x_grouped, W1, b1.reshape(E, 1, DFF), W2,
      b2.reshape(E, 1, C))

    out_tok = sc_gather(y_grouped, dest_flat)
    output = jnp.transpose(out_tok.reshape(Bc, Hh, Ww, Cc), (0, 3, 1, 2))
    kl_div = jnp.asarray(0.0, dtype=jnp.float32)
    return (output, kl_div, probs)


# speedup vs baseline: 1.8152x; 1.0502x over previous
import functools

import jax
import jax.numpy as jnp
from jax import lax
from jax.experimental import pallas as pl
from jax.experimental.pallas import tpu as pltpu
from jax.experimental.pallas import tpu_sc as plsc

N = 3136
C = 768
E = 8
DFF = 3072
T = 256
NB = 21
NPAD = NB * T
RB = 392
RG = N // RB
NW = 32
CH = 104
NMETA = 24


def _dispatch_kernel(x_ref, wr_ref, fm_ref, tm_ref, cm_ref,
                     probs_ref, dest_ref, meta_ref,
                     carry_ref, oh_ref, rank_ref):
    b = pl.program_id(0)

    @pl.when(b == 0)
    def _():
        carry_ref[...] = jnp.zeros_like(carry_ref)

    @pl.when(b < RG)
    def _():
        fp = jnp.dot(x_ref[...], fm_ref[...],
                     preferred_element_type=jnp.float32)
        tp = jnp.dot(wr_ref[...], tm_ref[...],
                     preferred_element_type=jnp.float32)
        comb = jnp.concatenate([fp, tp], axis=1)
        logits = jnp.dot(comb, cm_ref[...],
                         preferred_element_type=jnp.float32)
        m = jnp.max(logits, axis=1, keepdims=True)
        col = lax.broadcasted_iota(jnp.int32, (RB, E), 1)
        cand = jnp.where(logits == m, col, E)
        top = jnp.min(cand, axis=1, keepdims=True)
        oh = (col == top).astype(jnp.float32)
        probs_ref[...] = oh
        ri = lax.broadcasted_iota(jnp.int32, (RB, RB), 0)
        rj = lax.broadcasted_iota(jnp.int32, (RB, RB), 1)
        tri = (ri > rj).astype(jnp.float32)
        cum = jnp.dot(tri, oh, preferred_element_type=jnp.float32)
        carry = carry_ref[...]
        oh_ref[pl.ds(b * RB, RB), :] = oh
        rank_ref[pl.ds(b * RB, RB), :] = jnp.sum(oh * (cum + carry), axis=1,
                                                 keepdims=True)
        carry_ref[...] = carry + jnp.sum(oh, axis=0, keepdims=True)

    @pl.when(b == RG)
    def _():
        c = carry_ref[...].astype(jnp.int32)
        pc = (((c + (T - 1)) // T) * T).astype(jnp.float32)
        pcm = jnp.broadcast_to(pc, (E, E))
        i8 = lax.broadcasted_iota(jnp.int32, (E, E), 0)
        j8 = lax.broadcasted_iota(jnp.int32, (E, E), 1)
        poff_col = jnp.sum(jnp.where(j8 < i8, pcm, 0.0), axis=1,
                           keepdims=True)
        cum_row = jnp.dot(pc, (i8 <= j8).astype(jnp.float32),
                          preferred_element_type=jnp.float32)
        dest = jnp.dot(oh_ref[...], poff_col,
                       preferred_element_type=jnp.float32) + rank_ref[...]
        dest_ref[...] = dest.astype(jnp.int32)
        thr = (lax.broadcasted_iota(jnp.int32, (NMETA, 1), 0)
               * T).astype(jnp.float32)
        be = jnp.sum((jnp.broadcast_to(cum_row, (NMETA, E)) <=
                      jnp.broadcast_to(thr, (NMETA, E))).astype(jnp.float32),
                     axis=1, keepdims=True)
        be = jnp.minimum(be, float(E - 1))
        nact = jnp.broadcast_to(cum_row[:, E - 1:] * (1.0 / T), (NMETA, 1))
        rows = lax.broadcasted_iota(jnp.int32, (NMETA, 1), 0)
        meta_ref[...] = jnp.where(rows < NB, be, nact).astype(jnp.int32)


def _ffn_kernel(meta_ref, x_ref, w1_ref, b1_ref, w2_ref, b2_ref, o_ref):
    bb = pl.program_id(0)
    nact = meta_ref[NB, 0]

    @pl.when(bb < nact)
    def _():
        h = jnp.dot(x_ref[...], w1_ref[0], preferred_element_type=jnp.float32)
        h = h + b1_ref[0]
        g = h * 0.5 * (1.0 + lax.erf(h * 0.7071067811865476))
        o_ref[...] = (jnp.dot(g, w2_ref[0], preferred_element_type=jnp.float32)
                      + b2_ref[0])


@functools.cache
def _sc_kernels():
    mesh = plsc.VectorSubcoreMesh(core_axis_name="c", subcore_axis_name="s")
    TAIL = N - 30 * CH

    @functools.partial(
        pl.kernel, mesh=mesh,
        out_type=jax.ShapeDtypeStruct((NPAD, C), jnp.float32),
        scratch_types=[pltpu.VMEM((CH,), jnp.int32),
                       pltpu.VMEM((CH, C), jnp.float32),
                       pltpu.VMEM((TAIL,), jnp.int32),
                       pltpu.VMEM((TAIL, C), jnp.float32),
                       pltpu.SemaphoreType.DMA],
    )
    def sc_scatter(src_hbm, idx_hbm, out_hbm, idx_v, rows_v, idx_t, rows_t,
                   sem):
        wid = lax.axis_index("s") * 2 + lax.axis_index("c")
        base = wid * CH

        @pl.when(wid < 30)
        def _():
            pltpu.sync_copy(src_hbm.at[pl.ds(base, CH)], rows_v)
            pltpu.sync_copy(idx_hbm.at[pl.ds(base, CH)], idx_v)
            pltpu.async_copy(rows_v, out_hbm.at[idx_v], sem).wait()

        @pl.when(wid == 30)
        def _():
            pltpu.sync_copy(src_hbm.at[pl.ds(30 * CH, TAIL)], rows_t)
            pltpu.sync_copy(idx_hbm.at[pl.ds(30 * CH, TAIL)], idx_t)
            pltpu.async_copy(rows_t, out_hbm.at[idx_t], sem).wait()

    @functools.partial(
        pl.kernel, mesh=mesh,
        out_type=jax.ShapeDtypeStruct((N, C), jnp.float32),
        scratch_types=[pltpu.VMEM((CH,), jnp.int32),
                       pltpu.VMEM((CH, C), jnp.float32),
                       pltpu.VMEM((TAIL,), jnp.int32),
                       pltpu.VMEM((TAIL, C), jnp.float32),
                       pltpu.SemaphoreType.DMA],
    )
    def sc_gather(src_hbm, idx_hbm, out_hbm, idx_v, rows_v, idx_t, rows_t,
                  sem):
        wid = lax.axis_index("s") * 2 + lax.axis_index("c")
        base = wid * CH

        @pl.when(wid < 30)
        def _():
            pltpu.sync_copy(idx_hbm.at[pl.ds(base, CH)], idx_v)
            pltpu.async_copy(src_hbm.at[idx_v], rows_v, sem).wait()
            pltpu.sync_copy(rows_v, out_hbm.at[pl.ds(base, CH)])

        @pl.when(wid == 30)
        def _():
            pltpu.sync_copy(idx_hbm.at[pl.ds(30 * CH, TAIL)], idx_t)
            pltpu.async_copy(src_hbm.at[idx_t], rows_t, sem).wait()
            pltpu.sync_copy(rows_t, out_hbm.at[pl.ds(30 * CH, TAIL)])

    return sc_scatter, sc_gather


def kernel(x, w, feature_mu, text_mu, combined_mu, temperature, W1, b1, W2, b2):
    del temperature
    Bc, Cc, Hh, Ww = x.shape
    x_tok = jnp.transpose(x, (0, 2, 3, 1)).reshape(N, C)
    w_r = jnp.broadcast_to(w[:, None, None, :],
                           (Bc, Hh, Ww, w.shape[1])).reshape(N, w.shape[1])

    probs, dest, meta = pl.pallas_call(
        _dispatch_kernel,
        grid=(RG + 1,),
        in_specs=[
            pl.BlockSpec((RB, C), lambda b: (jnp.minimum(b, RG - 1), 0)),
            pl.BlockSpec((RB, 512), lambda b: (jnp.minimum(b, RG - 1), 0)),
            pl.BlockSpec((C, 128), lambda b: (0, 0)),
            pl.BlockSpec((512, 128), lambda b: (0, 0)),
            pl.BlockSpec((256, E), lambda b: (0, 0)),
        ],
        out_specs=[
            pl.BlockSpec((RB, E), lambda b: (jnp.minimum(b, RG - 1), 0)),
            pl.BlockSpec((N, 1), lambda b: (0, 0)),
            pl.BlockSpec((NMETA, 1), lambda b: (0, 0)),
        ],
        out_shape=[
            jax.ShapeDtypeStruct((N, E), jnp.float32),
            jax.ShapeDtypeStruct((N, 1), jnp.int32),
            jax.ShapeDtypeStruct((NMETA, 1), jnp.int32),
        ],
        scratch_shapes=[pltpu.VMEM((1, E), jnp.float32),
                        pltpu.VMEM((N, E), jnp.float32),
                        pltpu.VMEM((N, 1), jnp.float32)],
    )(x_tok, w_r, feature_mu, text_mu, combined_mu)

    sc_scatter, sc_gather = _sc_kernels()
    dest_flat = dest.reshape(N)
    x_grouped = sc_scatter(x_tok, dest_flat)

    y_grouped = pl.pallas_call(
        _ffn_kernel,
        grid_spec=pltpu.PrefetchScalarGridSpec(
            num_scalar_prefetch=1,
            grid=(NB,),
            in_specs=[
                pl.BlockSpec((T, C), lambda bb, s: (bb, 0)),
                pl.BlockSpec((1, C, DFF), lambda bb, s: (s[bb, 0], 0, 0)),
                pl.BlockSpec((1, 1, DFF), lambda bb, s: (s[bb, 0], 0, 0)),
                pl.BlockSpec((1, DFF, C), lambda bb, s: (s[bb, 0], 0, 0)),
                pl.BlockSpec((1, 1, C), lambda bb, s: (s[bb, 0], 0, 0)),
            ],
            out_specs=pl.BlockSpec((T, C), lambda bb, s: (bb, 0)),
        ),
        out_shape=jax.ShapeDtypeStruct((NPAD, C), jnp.float32),
    )(meta, x_grouped, W1, b1.reshape(E, 1, DFF), W2,
      b2.reshape(E, 1, C))

    out_tok = sc_gather(y_grouped, dest_flat)
    output = jnp.transpose(out_tok.reshape(Bc, Hh, Ww, Cc), (0, 3, 1, 2))
    kl_div = jnp.asarray(0.0, dtype=jnp.float32)
    return (output, kl_div, probs)
